# Initial kernel scaffold; baseline (speedup 1.0000x reference)
import jax
import jax.numpy as jnp
from jax.experimental import pallas as pl


def kernel(logit, feat, w1, b1, w2, b2):
    raise NotImplementedError("write your pallas kernel here")


# speedup vs baseline: 1.6651x; 1.6651x over previous
import functools

import jax
import jax.numpy as jnp
from jax.experimental import pallas as pl
from jax.experimental.pallas import tpu as pltpu

_NPT = 8192
_CL = 90
_CP = 96
_INTERPRET = False


def _up_unc_body(up_ref, c_ref, dn_ref, out_ref, unc_ref):
    i = pl.program_id(1)
    nb = pl.num_programs(1)
    c = c_ref[0]
    R, Win, CP = c.shape
    prev_last = jnp.where(i == 0, c[0:1], up_ref[0][R - 1:R])
    next_first = jnp.where(i == nb - 1, c[R - 1:R], dn_ref[0][0:1])
    km1 = jnp.concatenate([prev_last, c[:-1]], axis=0)
    kp1 = jnp.concatenate([c[1:], next_first], axis=0)
    even = 0.25 * km1 + 0.75 * c
    odd = 0.75 * c + 0.25 * kp1
    rows = jnp.stack([even, odd], axis=1).reshape(2 * R, Win, CP)
    cm1 = jnp.concatenate([rows[:, :1], rows[:, :-1]], axis=1)
    cp1 = jnp.concatenate([rows[:, 1:], rows[:, -1:]], axis=1)
    ev = 0.25 * cm1 + 0.75 * rows
    od = 0.75 * rows + 0.25 * cp1
    out = jnp.stack([ev, od], axis=2).reshape(2 * R, 2 * Win, CP)
    out_ref[0] = out
    lane = jax.lax.broadcasted_iota(jnp.int32, out.shape, 2)
    lm = jnp.where(lane < _CL, out, -1e30)
    m1 = jnp.max(lm, axis=2, keepdims=True)
    eqc = jnp.sum((lm == m1).astype(jnp.float32), axis=2)
    m2 = jnp.max(jnp.where(lm >= m1, -1e30, lm), axis=2)
    m2 = jnp.where(eqc > 1.0, m1[..., 0], m2)
    z = jnp.sum(jnp.exp(lm - m1), axis=2)
    unc_ref[0] = (jnp.exp(m2 - m1[..., 0]) - 1.0) / z


def _upsample_unc(x, R):
    B, Hin, Win, CP = x.shape
    nb = Hin // R
    return pl.pallas_call(
        _up_unc_body,
        grid=(B, nb),
        in_specs=[
            pl.BlockSpec((1, R, Win, CP), lambda b, i: (b, jnp.maximum(i - 1, 0), 0, 0)),
            pl.BlockSpec((1, R, Win, CP), lambda b, i: (b, i, 0, 0)),
            pl.BlockSpec((1, R, Win, CP),
                         lambda b, i, nb=nb: (b, jnp.minimum(i + 1, nb - 1), 0, 0)),
        ],
        out_specs=[
            pl.BlockSpec((1, 2 * R, 2 * Win, CP), lambda b, i: (b, i, 0, 0)),
            pl.BlockSpec((1, 2 * R, 2 * Win), lambda b, i: (b, i, 0)),
        ],
        out_shape=[
            jax.ShapeDtypeStruct((B, 2 * Hin, 2 * Win, CP), jnp.float32),
            jax.ShapeDtypeStruct((B, 2 * Hin, 2 * Win), jnp.float32),
        ],
        interpret=_INTERPRET,
    )(x, x, x)


def _mlp_body(f00, f01, f10, f11, wts, w1, b1, w2, b2, out_ref):
    w = wts[0]
    x = (f00[0] * w[:, 0:1] + f01[0] * w[:, 1:2]
         + f10[0] * w[:, 2:3] + f11[0] * w[:, 3:4])
    h = jax.lax.dot_general(x, w1[...], (((1,), (1,)), ((), ())),
                            preferred_element_type=jnp.float32) + b1[...][None, :]
    h = jnp.maximum(h, 0.0)
    o = jax.lax.dot_general(h, w2[...], (((1,), (1,)), ((), ())),
                            preferred_element_type=jnp.float32) + b2[...][None, :]
    Pb = o.shape[0]
    out_ref[0] = jnp.concatenate(
        [o, jnp.ones((Pb, 1), jnp.float32), jnp.zeros((Pb, _CP - _CL - 1), jnp.float32)],
        axis=1)


def _mlp(f00, f01, f10, f11, wts, w1, b1, w2, b2):
    B, P, Cf = f00.shape
    Pb = 2048
    grid = (B, P // Pb)
    fspec = pl.BlockSpec((1, Pb, Cf), lambda b, j: (b, j, 0))
    return pl.pallas_call(
        _mlp_body,
        grid=grid,
        in_specs=[
            fspec, fspec, fspec, fspec,
            pl.BlockSpec((1, Pb, 4), lambda b, j: (b, j, 0)),
            pl.BlockSpec(w1.shape, lambda b, j: (0, 0)),
            pl.BlockSpec(b1.shape, lambda b, j: (0,)),
            pl.BlockSpec(w2.shape, lambda b, j: (0, 0)),
            pl.BlockSpec(b2.shape, lambda b, j: (0,)),
        ],
        out_specs=pl.BlockSpec((1, Pb, _CP), lambda b, j: (b, j, 0)),
        out_shape=jax.ShapeDtypeStruct((B, P, _CP), jnp.float32),
        interpret=_INTERPRET,
    )(f00, f01, f10, f11, wts, w1, b1, w2, b2)


def kernel(logit, feat, w1, b1, w2, b2):
    B, C, H, W = logit.shape
    Cf, Hf, Wf = feat.shape[1], feat.shape[2], feat.shape[3]
    x = jnp.transpose(logit, (0, 2, 3, 1))
    x = jnp.pad(x, ((0, 0), (0, 0), (0, 0), (0, _CP - C)))
    featT = jnp.transpose(feat, (0, 2, 3, 1)).reshape(B, Hf * Wf, Cf)
    b_ix = jnp.arange(B)[:, None]

    for _ in range(3):
        Hin, Win = x.shape[1], x.shape[2]
        Hh, Ww = Hin * 2, Win * 2
        R = {48: 24, 96: 16, 192: 16}[Hin]
        up, unc = _upsample_unc(x, R)

        idx = jax.lax.top_k(unc.reshape(B, Hh * Ww), _NPT)[1].astype(jnp.int32)

        hh = (idx // Ww).astype(jnp.float32)
        ww = (idx % Ww).astype(jnp.float32)
        xf = (ww + 0.5) / Ww * Wf - 0.5
        yf = (hh + 0.5) / Hh * Hf - 0.5
        x0 = jnp.floor(xf)
        y0 = jnp.floor(yf)
        wx = xf - x0
        wy = yf - y0
        x0c = jnp.clip(x0, 0, Wf - 1).astype(jnp.int32)
        x1c = jnp.clip(x0 + 1, 0, Wf - 1).astype(jnp.int32)
        y0c = jnp.clip(y0, 0, Hf - 1).astype(jnp.int32)
        y1c = jnp.clip(y0 + 1, 0, Hf - 1).astype(jnp.int32)
        gat = lambda r: jnp.take_along_axis(featT, r[..., None], axis=1)
        f00 = gat(y0c * Wf + x0c)
        f01 = gat(y0c * Wf + x1c)
        f10 = gat(y1c * Wf + x0c)
        f11 = gat(y1c * Wf + x1c)
        wts = jnp.stack([(1 - wx) * (1 - wy), wx * (1 - wy),
                         (1 - wx) * wy, wx * wy], axis=-1)

        vals = _mlp(f00, f01, f10, f11, wts, w1, b1, w2, b2)

        uflat = up.reshape(B, Hh * Ww, _CP)
        uflat = uflat.at[b_ix, idx].set(vals)
        x = uflat.reshape(B, Hh, Ww, _CP)

    out_logit = jnp.transpose(x[..., :C], (0, 3, 1, 2))
    painter = x[..., _CL][:, None]
    return out_logit, painter


# speedup vs baseline: 2.1504x; 1.2914x over previous
import functools

import jax
import jax.numpy as jnp
from jax.experimental import pallas as pl
from jax.experimental.pallas import tpu as pltpu
from jax.experimental.pallas import tpu_sc as plsc

_NPT = 8192
_CL = 90
_CP = 128
_INTERPRET = False


def _up_unc_body(up_ref, c_ref, dn_ref, out_ref, unc_ref):
    i = pl.program_id(1)
    nb = pl.num_programs(1)
    c = c_ref[0]
    R, Win, CP = c.shape
    prev_last = jnp.where(i == 0, c[0:1], up_ref[0][R - 1:R])
    next_first = jnp.where(i == nb - 1, c[R - 1:R], dn_ref[0][0:1])
    km1 = jnp.concatenate([prev_last, c[:-1]], axis=0)
    kp1 = jnp.concatenate([c[1:], next_first], axis=0)
    even = 0.25 * km1 + 0.75 * c
    odd = 0.75 * c + 0.25 * kp1
    rows = jnp.stack([even, odd], axis=1).reshape(2 * R, Win, CP)
    cm1 = jnp.concatenate([rows[:, :1], rows[:, :-1]], axis=1)
    cp1 = jnp.concatenate([rows[:, 1:], rows[:, -1:]], axis=1)
    ev = 0.25 * cm1 + 0.75 * rows
    od = 0.75 * rows + 0.25 * cp1
    out = jnp.stack([ev, od], axis=2).reshape(2 * R, 2 * Win, CP)
    out_ref[0] = out
    lane = jax.lax.broadcasted_iota(jnp.int32, out.shape, 2)
    lm = jnp.where(lane < _CL, out, -1e30)
    m1 = jnp.max(lm, axis=2, keepdims=True)
    eqc = jnp.sum((lm == m1).astype(jnp.float32), axis=2)
    m2 = jnp.max(jnp.where(lm >= m1, -1e30, lm), axis=2)
    m2 = jnp.where(eqc > 1.0, m1[..., 0], m2)
    z = jnp.sum(jnp.exp(lm - m1), axis=2)
    unc_ref[0] = (jnp.exp(m2 - m1[..., 0]) - 1.0) / z


def _upsample_unc(x, R):
    B, Hin, Win, CP = x.shape
    nb = Hin // R
    return pl.pallas_call(
        _up_unc_body,
        grid=(B, nb),
        in_specs=[
            pl.BlockSpec((1, R, Win, CP), lambda b, i: (b, jnp.maximum(i - 1, 0), 0, 0)),
            pl.BlockSpec((1, R, Win, CP), lambda b, i: (b, i, 0, 0)),
            pl.BlockSpec((1, R, Win, CP),
                         lambda b, i, nb=nb: (b, jnp.minimum(i + 1, nb - 1), 0, 0)),
        ],
        out_specs=[
            pl.BlockSpec((1, 2 * R, 2 * Win, CP), lambda b, i: (b, i, 0, 0)),
            pl.BlockSpec((1, 2 * R, 2 * Win), lambda b, i: (b, i, 0)),
        ],
        out_shape=[
            jax.ShapeDtypeStruct((B, 2 * Hin, 2 * Win, CP), jnp.float32),
            jax.ShapeDtypeStruct((B, 2 * Hin, 2 * Win), jnp.float32),
        ],
        interpret=_INTERPRET,
    )(x, x, x)


def _mlp_body(f00, f01, f10, f11, wts, w1t, b1, w2t, b2p, out_ref):
    w = wts[0]
    x = (f00[0] * w[:, 0:1] + f01[0] * w[:, 1:2]
         + f10[0] * w[:, 2:3] + f11[0] * w[:, 3:4])
    h = jnp.dot(x, w1t[...], preferred_element_type=jnp.float32) + b1[...][None, :]
    h = jnp.maximum(h, 0.0)
    out_ref[0] = (jnp.dot(h, w2t[...], preferred_element_type=jnp.float32)
                  + b2p[...][None, :])


def _mlp(f00, f01, f10, f11, wts, w1t, b1, w2t, b2p):
    B, P, Cf = f00.shape
    Pb = 2048
    grid = (B, P // Pb)
    fspec = pl.BlockSpec((1, Pb, Cf), lambda b, j: (b, j, 0))
    return pl.pallas_call(
        _mlp_body,
        grid=grid,
        in_specs=[
            fspec, fspec, fspec, fspec,
            pl.BlockSpec((1, Pb, 4), lambda b, j: (b, j, 0)),
            pl.BlockSpec(w1t.shape, lambda b, j: (0, 0)),
            pl.BlockSpec(b1.shape, lambda b, j: (0,)),
            pl.BlockSpec(w2t.shape, lambda b, j: (0, 0)),
            pl.BlockSpec(b2p.shape, lambda b, j: (0,)),
        ],
        out_specs=pl.BlockSpec((1, Pb, _CP), lambda b, j: (b, j, 0)),
        out_shape=jax.ShapeDtypeStruct((B, P, _CP), jnp.float32),
        interpret=_INTERPRET,
    )(f00, f01, f10, f11, wts, w1t, b1, w2t, b2p)


_NW = 32
_SC_CH = 128


def _sc_scatter(dst_ref, idx_g, vals):
    n = idx_g.shape[0]
    per_w = n // _NW
    nch = per_w // _SC_CH
    idx3 = idx_g.reshape(_NW, nch, _SC_CH)
    vals4 = vals.reshape(_NW, nch, _SC_CH, _CP)
    mesh = plsc.VectorSubcoreMesh(core_axis_name="c", subcore_axis_name="s")

    @functools.partial(
        pl.kernel,
        mesh=mesh,
        scratch_types=[
            pltpu.VMEM((nch, _SC_CH), jnp.int32),
            pltpu.VMEM((_SC_CH, _CP), jnp.float32),
            pltpu.SemaphoreType.DMA,
        ],
    )
    def body(idx_hbm, vals_hbm, dref, idx_v, vals_v, sem):
        wid = jax.lax.axis_index("s") * 2 + jax.lax.axis_index("c")
        pltpu.sync_copy(idx_hbm.at[wid], idx_v)
        for j in range(nch):
            pltpu.sync_copy(vals_hbm.at[wid, j], vals_v)
            pltpu.async_copy(vals_v, dref.at[idx_v.at[j]], sem).wait()

    body(idx3, vals4, dst_ref)


def kernel(logit, feat, w1, b1, w2, b2):
    B, C, H, W = logit.shape
    Cf, Hf, Wf = feat.shape[1], feat.shape[2], feat.shape[3]
    x = jnp.transpose(logit, (0, 2, 3, 1))
    x = jnp.pad(x, ((0, 0), (0, 0), (0, 0), (0, _CP - C)))
    featT = jnp.transpose(feat, (0, 2, 3, 1)).reshape(B, Hf * Wf, Cf)
    w1t = w1.T
    w2t = jnp.pad(w2, ((0, _CP - C), (0, 0))).T
    b2p = jnp.pad(b2, (0, _CP - C)).at[_CL].set(1.0)

    for _ in range(3):
        Hin, Win = x.shape[1], x.shape[2]
        Hh, Ww = Hin * 2, Win * 2
        R = {48: 24, 96: 16, 192: 16}[Hin]
        up, unc = _upsample_unc(x, R)

        idx = jax.lax.top_k(unc.reshape(B, Hh * Ww), _NPT)[1].astype(jnp.int32)

        hh = (idx // Ww).astype(jnp.float32)
        ww = (idx % Ww).astype(jnp.float32)
        xf = (ww + 0.5) / Ww * Wf - 0.5
        yf = (hh + 0.5) / Hh * Hf - 0.5
        x0 = jnp.floor(xf)
        y0 = jnp.floor(yf)
        wx = xf - x0
        wy = yf - y0
        x0c = jnp.clip(x0, 0, Wf - 1).astype(jnp.int32)
        x1c = jnp.clip(x0 + 1, 0, Wf - 1).astype(jnp.int32)
        y0c = jnp.clip(y0, 0, Hf - 1).astype(jnp.int32)
        y1c = jnp.clip(y0 + 1, 0, Hf - 1).astype(jnp.int32)
        gat = lambda r: jnp.take_along_axis(featT, r[..., None], axis=1)
        f00 = gat(y0c * Wf + x0c)
        f01 = gat(y0c * Wf + x1c)
        f10 = gat(y1c * Wf + x0c)
        f11 = gat(y1c * Wf + x1c)
        wts = jnp.stack([(1 - wx) * (1 - wy), wx * (1 - wy),
                         (1 - wx) * wy, wx * wy], axis=-1)

        vals = _mlp(f00, f01, f10, f11, wts, w1t, b1, w2t, b2p)

        idx_g = (idx + jnp.arange(B, dtype=jnp.int32)[:, None] * (Hh * Ww)).reshape(-1)
        dst_ref = jax.new_ref(up.reshape(B * Hh * Ww, _CP))
        _sc_scatter(dst_ref, idx_g, vals.reshape(B * _NPT, _CP))
        x = dst_ref[...].reshape(B, Hh, Ww, _CP)

    out_logit = jnp.transpose(x[..., :C], (0, 3, 1, 2))
    painter = x[..., _CL][:, None]
    return out_logit, painter
